# baseline (device time: 130871 ns/iter reference)
import functools

import jax
import jax.numpy as jnp
from jax import lax
from jax.experimental import pallas as pl
from jax.experimental.pallas import tpu as pltpu

N_DEV = 16
B_LOC = 2
SQ = 128
SKV = 128
HQ_PER = 4
DH = 64
D_MODEL = 512
CHUNK = HQ_PER * DH
ROWS = B_LOC * SQ
KROWS = B_LOC * SKV


def _body(x_ref, wq_ref, wo_ref, k_ref, v_ref, out_ref,
          wq_buf, wo_buf, sq_send, sq_recv, so_send, so_recv):
    my = lax.axis_index("i")
    right = lax.rem(my + 1, N_DEV)
    left = lax.rem(my + N_DEV - 1, N_DEV)

    barrier = pltpu.get_barrier_semaphore()
    for nbr in (left, right):
        pl.semaphore_signal(barrier, inc=1, device_id=(nbr,),
                            device_id_type=pl.DeviceIdType.MESH)
    pl.semaphore_wait(barrier, 2)

    wq_buf[0] = wq_ref[...]
    wo_buf[0] = wo_ref[...]
    out_ref[...] = jnp.zeros_like(out_ref)

    ri = lax.broadcasted_iota(jnp.int32, (ROWS, KROWS), 0) // SQ
    ci = lax.broadcasted_iota(jnp.int32, (ROWS, KROWS), 1) // SKV
    bias = jnp.where(ri == ci, 0.0, -1e9).astype(jnp.float32)

    xv = x_ref[...]

    def compute(slot):
        o = lax.rem(my - slot + N_DEV, N_DEV)
        wq_c = wq_buf[slot]
        wo_c = wo_buf[slot]
        qs = lax.dot_general(
            xv, wq_c, (((1,), (0,)), ((), ())),
            preferred_element_type=jnp.float32,
        ).astype(jnp.bfloat16)
        acc = jnp.zeros((ROWS, D_MODEL), jnp.float32)
        for h in range(HQ_PER):
            g = o * HQ_PER + h
            q_h = qs[:, h * DH:(h + 1) * DH]
            k_h = k_ref[g]
            s = lax.dot_general(
                q_h, k_h, (((1,), (1,)), ((), ())),
                preferred_element_type=jnp.float32,
            )
            s = s * 0.125 + bias
            m = jnp.max(s, axis=1, keepdims=True)
            w = jnp.exp(s - m)
            w = (w / jnp.sum(w, axis=1, keepdims=True)).astype(jnp.bfloat16)
            c = lax.dot_general(
                w, v_ref[g], (((1,), (0,)), ((), ())),
                preferred_element_type=jnp.float32,
            ).astype(jnp.bfloat16)
            acc = acc + lax.dot_general(
                c, wo_c[h * DH:(h + 1) * DH, :], (((1,), (0,)), ((), ())),
                preferred_element_type=jnp.float32,
            )
        out_ref[...] += acc

    def hop(h, carry):
        rq = pltpu.make_async_remote_copy(
            src_ref=wq_buf.at[h], dst_ref=wq_buf.at[h + 1],
            send_sem=sq_send.at[h], recv_sem=sq_recv.at[h],
            device_id=(right,), device_id_type=pl.DeviceIdType.MESH,
        )
        rq.start()
        ro = pltpu.make_async_remote_copy(
            src_ref=wo_buf.at[h], dst_ref=wo_buf.at[h + 1],
            send_sem=so_send.at[h], recv_sem=so_recv.at[h],
            device_id=(right,), device_id_type=pl.DeviceIdType.MESH,
        )
        ro.start()
        compute(h)
        rq.wait_recv()
        ro.wait_recv()
        return carry

    lax.fori_loop(0, N_DEV - 1, hop, 0)
    compute(N_DEV - 1)

    def drain(h, carry):
        dq = pltpu.make_async_remote_copy(
            src_ref=wq_buf.at[h], dst_ref=wq_buf.at[h],
            send_sem=sq_send.at[h], recv_sem=sq_recv.at[h],
            device_id=(right,), device_id_type=pl.DeviceIdType.MESH,
        )
        dq.wait_send()
        do = pltpu.make_async_remote_copy(
            src_ref=wo_buf.at[h], dst_ref=wo_buf.at[h],
            send_sem=so_send.at[h], recv_sem=so_recv.at[h],
            device_id=(right,), device_id_type=pl.DeviceIdType.MESH,
        )
        do.wait_send()
        return carry

    lax.fori_loop(0, N_DEV - 1, drain, 0)

    @functools.partial(pl.run_scoped, sem2=pltpu.SemaphoreType.REGULAR)
    def _(sem2):
        for nbr in (left, right):
            pl.semaphore_signal(sem2, inc=1, device_id=(nbr,),
                                device_id_type=pl.DeviceIdType.MESH)
        pl.semaphore_wait(sem2, 2)


def kernel(x, Wq, K_ext, V_ext, Wo):
    my = lax.axis_index("i")
    K_loc = lax.dynamic_slice_in_dim(K_ext, my * B_LOC, B_LOC, axis=0)
    V_loc = lax.dynamic_slice_in_dim(V_ext, my * B_LOC, B_LOC, axis=0)
    k_st = K_loc.transpose(2, 0, 1, 3).reshape(
        N_DEV * HQ_PER, KROWS, DH).astype(jnp.bfloat16)
    v_st = V_loc.transpose(2, 0, 1, 3).reshape(
        N_DEV * HQ_PER, KROWS, DH).astype(jnp.bfloat16)
    x2d = x.reshape(ROWS, D_MODEL).astype(jnp.bfloat16)
    wq = Wq.astype(jnp.bfloat16)
    wo = Wo.astype(jnp.bfloat16)

    out2d = pl.pallas_call(
        _body,
        out_shape=jax.ShapeDtypeStruct((ROWS, D_MODEL), jnp.float32),
        in_specs=[pl.BlockSpec(memory_space=pltpu.VMEM)] * 5,
        out_specs=pl.BlockSpec(memory_space=pltpu.VMEM),
        scratch_shapes=[
            pltpu.VMEM((N_DEV, D_MODEL, CHUNK), jnp.bfloat16),
            pltpu.VMEM((N_DEV, CHUNK, D_MODEL), jnp.bfloat16),
            pltpu.SemaphoreType.DMA((N_DEV - 1,)),
            pltpu.SemaphoreType.DMA((N_DEV - 1,)),
            pltpu.SemaphoreType.DMA((N_DEV - 1,)),
            pltpu.SemaphoreType.DMA((N_DEV - 1,)),
        ],
        compiler_params=pltpu.CompilerParams(collective_id=0),
    )(x2d, wq, wo, k_st, v_st)
    return out2d.reshape(B_LOC, SQ, D_MODEL)


# device time: 76377 ns/iter; 1.7135x vs baseline; 1.7135x over previous
import functools

import jax
import jax.numpy as jnp
from jax import lax
from jax.experimental import pallas as pl
from jax.experimental.pallas import tpu as pltpu

N_DEV = 16
B_LOC = 2
SQ = 128
SKV = 128
HQ_PER = 4
DH = 64
D_MODEL = 512
CHUNK = HQ_PER * DH
ROWS = B_LOC * SQ
KROWS = B_LOC * SKV

CW_HOPS = N_DEV // 2
CCW_HOPS = N_DEV // 2 - 1


def _body(x_ref, wq_ref, wo_ref, k_ref, v_ref, out_ref,
          wq_buf, wo_buf,
          qcw_s, qcw_r, ocw_s, ocw_r,
          qcc_s, qcc_r, occ_s, occ_r):
    my = lax.axis_index("i")
    right = lax.rem(my + 1, N_DEV)
    left = lax.rem(my + N_DEV - 1, N_DEV)

    barrier = pltpu.get_barrier_semaphore()
    for nbr in (left, right):
        pl.semaphore_signal(barrier, inc=1, device_id=(nbr,),
                            device_id_type=pl.DeviceIdType.MESH)
    pl.semaphore_wait(barrier, 2)

    wq_buf[0] = wq_ref[...]
    wo_buf[0] = wo_ref[...]
    out_ref[...] = jnp.zeros_like(out_ref)

    ri = lax.broadcasted_iota(jnp.int32, (ROWS, KROWS), 0) // SQ
    ci = lax.broadcasted_iota(jnp.int32, (ROWS, KROWS), 1) // SKV
    bias = jnp.where(ri == ci, 0.0, -1e9).astype(jnp.float32)

    xv = x_ref[...]

    def compute(slot):
        o = lax.rem(my - slot + 2 * N_DEV, N_DEV)
        wq_c = wq_buf[slot]
        wo_c = wo_buf[slot]
        qs = lax.dot_general(
            xv, wq_c, (((1,), (0,)), ((), ())),
            preferred_element_type=jnp.float32,
        ).astype(jnp.bfloat16)
        acc = jnp.zeros((ROWS, D_MODEL), jnp.float32)
        for h in range(HQ_PER):
            g = o * HQ_PER + h
            q_h = qs[:, h * DH:(h + 1) * DH]
            k_h = k_ref[g]
            s = lax.dot_general(
                q_h, k_h, (((1,), (1,)), ((), ())),
                preferred_element_type=jnp.float32,
            )
            s = s * 0.125 + bias
            m = jnp.max(s, axis=1, keepdims=True)
            w = jnp.exp(s - m)
            w = (w / jnp.sum(w, axis=1, keepdims=True)).astype(jnp.bfloat16)
            c = lax.dot_general(
                w, v_ref[g], (((1,), (0,)), ((), ())),
                preferred_element_type=jnp.float32,
            ).astype(jnp.bfloat16)
            acc = acc + lax.dot_general(
                c, wo_c[h * DH:(h + 1) * DH, :], (((1,), (0,)), ((), ())),
                preferred_element_type=jnp.float32,
            )
        out_ref[...] += acc

    def rdma(buf, src_slot, dst_slot, send_sems, recv_sems, idx, target):
        return pltpu.make_async_remote_copy(
            src_ref=buf.at[src_slot], dst_ref=buf.at[dst_slot],
            send_sem=send_sems.at[idx], recv_sem=recv_sems.at[idx],
            device_id=(target,), device_id_type=pl.DeviceIdType.MESH,
        )

    def step(h, carry):
        hm1 = jnp.maximum(h - 1, 0)
        hcw = jnp.minimum(h, CW_HOPS - 1)
        hcc = jnp.minimum(h, CCW_HOPS - 1)
        ccw_src = lax.rem(2 * N_DEV - h, N_DEV)
        ccw_slot = N_DEV - jnp.clip(h, 1, CCW_HOPS)

        @pl.when(h >= 1)
        def _():
            rdma(wq_buf, hm1, hm1 + 1, qcw_s, qcw_r, hm1, left).wait_recv()

        @pl.when(h < CW_HOPS)
        def _():
            rdma(wq_buf, hcw, hcw + 1, qcw_s, qcw_r, hcw, right).start()

        @pl.when(h >= 1)
        def _():
            rdma(wo_buf, hm1, hm1 + 1, ocw_s, ocw_r, hm1, left).wait_recv()

        @pl.when(h < CW_HOPS)
        def _():
            rdma(wo_buf, hcw, hcw + 1, ocw_s, ocw_r, hcw, right).start()

        @pl.when(jnp.logical_and(h >= 1, h <= CCW_HOPS))
        def _():
            rdma(wq_buf, 0, ccw_slot, qcc_s, qcc_r, hm1, right).wait_recv()

        @pl.when(h < CCW_HOPS)
        def _():
            rdma(wq_buf, ccw_src, N_DEV - 1 - hcc, qcc_s, qcc_r, hcc,
                 left).start()

        @pl.when(jnp.logical_and(h >= 1, h <= CCW_HOPS))
        def _():
            rdma(wo_buf, 0, ccw_slot, occ_s, occ_r, hm1, right).wait_recv()

        @pl.when(h < CCW_HOPS)
        def _():
            rdma(wo_buf, ccw_src, N_DEV - 1 - hcc, occ_s, occ_r, hcc,
                 left).start()

        @pl.when(h == 0)
        def _():
            compute(0)

        @pl.when(h >= 1)
        def _():
            compute(jnp.minimum(h, CW_HOPS))

        @pl.when(jnp.logical_and(h >= 1, h <= CCW_HOPS))
        def _():
            compute(ccw_slot)
        return carry

    lax.fori_loop(0, CW_HOPS + 1, step, 0)

    def drain(h, carry):
        hcc = jnp.minimum(h, CCW_HOPS - 1)
        rdma(wq_buf, h, h, qcw_s, qcw_r, h, right).wait_send()
        rdma(wo_buf, h, h, ocw_s, ocw_r, h, right).wait_send()

        @pl.when(h < CCW_HOPS)
        def _():
            rdma(wq_buf, hcc, hcc, qcc_s, qcc_r, hcc, left).wait_send()
            rdma(wo_buf, hcc, hcc, occ_s, occ_r, hcc, left).wait_send()
        return carry

    lax.fori_loop(0, CW_HOPS, drain, 0)

    @functools.partial(pl.run_scoped, sem2=pltpu.SemaphoreType.REGULAR)
    def _(sem2):
        for nbr in (left, right):
            pl.semaphore_signal(sem2, inc=1, device_id=(nbr,),
                                device_id_type=pl.DeviceIdType.MESH)
        pl.semaphore_wait(sem2, 2)


def kernel(x, Wq, K_ext, V_ext, Wo):
    my = lax.axis_index("i")
    K_loc = lax.dynamic_slice_in_dim(K_ext, my * B_LOC, B_LOC, axis=0)
    V_loc = lax.dynamic_slice_in_dim(V_ext, my * B_LOC, B_LOC, axis=0)
    k_st = K_loc.transpose(2, 0, 1, 3).reshape(
        N_DEV * HQ_PER, KROWS, DH).astype(jnp.bfloat16)
    v_st = V_loc.transpose(2, 0, 1, 3).reshape(
        N_DEV * HQ_PER, KROWS, DH).astype(jnp.bfloat16)
    x2d = x.reshape(ROWS, D_MODEL).astype(jnp.bfloat16)
    wq = Wq.astype(jnp.bfloat16)
    wo = Wo.astype(jnp.bfloat16)

    out2d = pl.pallas_call(
        _body,
        out_shape=jax.ShapeDtypeStruct((ROWS, D_MODEL), jnp.float32),
        in_specs=[pl.BlockSpec(memory_space=pltpu.VMEM)] * 5,
        out_specs=pl.BlockSpec(memory_space=pltpu.VMEM),
        scratch_shapes=[
            pltpu.VMEM((N_DEV, D_MODEL, CHUNK), jnp.bfloat16),
            pltpu.VMEM((N_DEV, CHUNK, D_MODEL), jnp.bfloat16),
            pltpu.SemaphoreType.DMA((CW_HOPS,)),
            pltpu.SemaphoreType.DMA((CW_HOPS,)),
            pltpu.SemaphoreType.DMA((CW_HOPS,)),
            pltpu.SemaphoreType.DMA((CW_HOPS,)),
            pltpu.SemaphoreType.DMA((CCW_HOPS,)),
            pltpu.SemaphoreType.DMA((CCW_HOPS,)),
            pltpu.SemaphoreType.DMA((CCW_HOPS,)),
            pltpu.SemaphoreType.DMA((CCW_HOPS,)),
        ],
        compiler_params=pltpu.CompilerParams(collective_id=0),
    )(x2d, wq, wo, k_st, v_st)
    return out2d.reshape(B_LOC, SQ, D_MODEL)
